# baseline (speedup 1.0000x reference)
import functools

import jax
import jax.numpy as jnp
from jax import lax
from jax.experimental import pallas as pl
from jax.experimental.pallas import tpu as pltpu
from jax.experimental.pallas import tpu_sc as plsc

B = 16384
M_DIM = 9
C = 32
NNZ = 90
ROW = M_DIM * C
NC, NS, L = 2, 16, 16
NW = NC * NS
RW = B // NW
R = 64
NCHUNK = RW // R
NJ = NNZ * C // L
CW = R * ROW


def _sc_body(x1_hbm, x2_hbm, i1_hbm, i2_hbm, io_hbm, cg_hbm, out_hbm,
             x1c, x2c, outc, i1v, i2v, iov, cgv, b1s, b2s, bos, cgs):
    wid = lax.axis_index("s") * NC + lax.axis_index("c")
    base = wid * (RW * ROW)
    pltpu.sync_copy(i1_hbm, i1v)
    pltpu.sync_copy(i2_hbm, i2v)
    pltpu.sync_copy(io_hbm, iov)
    pltpu.sync_copy(cg_hbm, cgv)

    def pbody(j, c):
        b1s[j] = jnp.min(i1v[pl.ds(j * L, L)])
        b2s[j] = jnp.min(i2v[pl.ds(j * L, L)])
        bos[j] = jnp.min(iov[pl.ds(j * L, L)])
        cgs[j] = jnp.min(cgv[pl.ds(j * L, L)])
        return c
    lax.fori_loop(0, NJ, pbody, 0)

    def chunk_body(ci, carry):
        off = base + ci * CW
        pltpu.sync_copy(x1_hbm.at[pl.ds(off, CW)], x1c)
        pltpu.sync_copy(x2_hbm.at[pl.ds(off, CW)], x2c)

        zero = jnp.zeros((L,), jnp.float32)

        @plsc.parallel_loop(0, CW // L, step=1, unroll=8)
        def zbody(q):
            outc[pl.ds(q * L, L)] = zero

        def jbody(j, c):
            o1 = b1s[j]
            o2 = b2s[j]
            oo = bos[j]
            cgb = jnp.full((L,), cgs[j], jnp.float32)

            @plsc.parallel_loop(0, CW, step=ROW, unroll=8)
            def rbody(r):
                a = x1c[pl.ds(r + o1, L)]
                b = x2c[pl.ds(r + o2, L)]
                plsc.addupdate(outc.at[pl.ds(r + oo, L)], a * b * cgb)
            return c
        lax.fori_loop(0, NJ, jbody, 0)

        pltpu.sync_copy(outc, out_hbm.at[pl.ds(off, CW)])
        return carry
    lax.fori_loop(0, NCHUNK, chunk_body, 0)


def kernel(x1, x2, CG_vals, M1, M2, M_ptr):
    seg_lens = M_ptr[1:] - M_ptr[:-1]
    seg_ids = jnp.repeat(
        jnp.arange(M_DIM, dtype=jnp.int32), seg_lens, total_repeat_length=NNZ
    )
    lanes = jnp.arange(C, dtype=jnp.int32)[None, :]
    i1 = (M1[:, None] * C + lanes).reshape(NJ * L)
    i2 = (M2[:, None] * C + lanes).reshape(NJ * L)
    io = (seg_ids[:, None] * C + lanes).reshape(NJ * L)
    cg = jnp.broadcast_to(CG_vals[:, None], (NNZ, C)).reshape(NJ * L)

    x1f = x1.reshape(B * ROW)
    x2f = x2.reshape(B * ROW)

    mesh = plsc.VectorSubcoreMesh(
        core_axis_name="c", subcore_axis_name="s", num_cores=NC, num_subcores=NS
    )
    out = pl.kernel(
        _sc_body,
        out_type=jax.ShapeDtypeStruct((B * ROW,), jnp.float32),
        mesh=mesh,
        compiler_params=pltpu.CompilerParams(needs_layout_passes=False),
        scratch_types=[
            pltpu.VMEM((CW,), jnp.float32),
            pltpu.VMEM((CW,), jnp.float32),
            pltpu.VMEM((CW,), jnp.float32),
            pltpu.VMEM((NJ * L,), jnp.int32),
            pltpu.VMEM((NJ * L,), jnp.int32),
            pltpu.VMEM((NJ * L,), jnp.int32),
            pltpu.VMEM((NJ * L,), jnp.float32),
            pltpu.SMEM((NJ,), jnp.int32),
            pltpu.SMEM((NJ,), jnp.int32),
            pltpu.SMEM((NJ,), jnp.int32),
            pltpu.SMEM((NJ,), jnp.float32),
        ],
    )(x1f, x2f, i1, i2, io, cg)
    return out.reshape(B, M_DIM, C)
